# baseline (speedup 1.0000x reference)
import functools

import jax
import jax.numpy as jnp
from jax import lax
from jax.experimental import pallas as pl
from jax.experimental.pallas import tpu as pltpu
from jax.experimental.pallas import tpu_sc as plsc

_B, _T, _D_IN, _D_LAT, _K = 128, 1024, 96, 32, 512
_N = _B * _T
_TB = 4096
_NT = _N // _TB

_DP = 128
_NW = 32
_BPW = _N // _NW
_CH = 128
_NCH = _BPW // _CH


def _prep_body(cb_ref, decw_ref, decb_ref, dect_ref, cm2_ref, c2_ref):
    cb = cb_ref[...]
    dect_ref[...] = (jnp.dot(cb, decw_ref[...],
                             preferred_element_type=jnp.float32)
                     + decb_ref[...])
    cm2_ref[...] = -2.0 * cb
    c2_ref[...] = jnp.sum(cb * cb, axis=1, keepdims=True)


def _prep_pass(codebook, dec_W, dec_b):
    return pl.pallas_call(
        _prep_body,
        out_shape=[
            jax.ShapeDtypeStruct((_K, _DP), jnp.float32),
            jax.ShapeDtypeStruct((_K, _D_LAT), jnp.float32),
            jax.ShapeDtypeStruct((_K, 1), jnp.float32),
        ],
    )(codebook, dec_W, dec_b.reshape(1, _DP))


def _vq_body(x_ref, encw_ref, encb_ref, cm2_ref, c2_ref, q_ref, loss_ref):
    i = pl.program_id(0)

    @pl.when(i == 0)
    def _init():
        loss_ref[...] = jnp.zeros((1, 1), jnp.float32)

    x = x_ref[...]
    zT = (lax.dot_general(encw_ref[...], x, (((0,), (1,)), ((), ())),
                          preferred_element_type=jnp.float32)
          + encb_ref[...])
    sT = lax.dot_general(cm2_ref[...], zT, (((1,), (0,)), ((), ())),
                         preferred_element_type=jnp.float32)
    dT = sT + c2_ref[...]
    dminT = jnp.min(dT, axis=0, keepdims=True)
    q = jnp.argmin(dT, axis=0).astype(jnp.int32)
    z2T = jnp.sum(zT * zT, axis=0, keepdims=True)
    loss_ref[...] += (jnp.sum(dminT + z2T, keepdims=True)
                      * (1.0 / (_N * _D_LAT)))
    q_ref[...] = q


def _tc_pass(x, enc_W, enc_b, cm2, c2):
    full = lambda i: (0, 0)
    return pl.pallas_call(
        _vq_body,
        grid=(_NT,),
        in_specs=[
            pl.BlockSpec((_TB, _D_IN), lambda i: (i, 0)),
            pl.BlockSpec((_D_IN, _D_LAT), full),
            pl.BlockSpec((_D_LAT, 1), full),
            pl.BlockSpec((_K, _D_LAT), full),
            pl.BlockSpec((_K, 1), full),
        ],
        out_specs=[
            pl.BlockSpec((_TB,), lambda i: (i,)),
            pl.BlockSpec((1, 1), full),
        ],
        out_shape=[
            jax.ShapeDtypeStruct((_N,), jnp.int32),
            jax.ShapeDtypeStruct((1, 1), jnp.float32),
        ],
    )(x, enc_W, enc_b.reshape(_D_LAT, 1), cm2, c2)


@functools.cache
def _make_sc_gather():
    mesh = plsc.VectorSubcoreMesh(core_axis_name="c", subcore_axis_name="s")

    @functools.partial(
        pl.kernel,
        mesh=mesh,
        out_type=jax.ShapeDtypeStruct((_N, _DP), jnp.float32),
        scratch_types=[
            pltpu.VMEM((_BPW,), jnp.int32),
            pltpu.VMEM((4, _CH, _DP), jnp.float32),
            pltpu.SemaphoreType.DMA,
            pltpu.SemaphoreType.DMA,
            pltpu.SemaphoreType.DMA,
            pltpu.SemaphoreType.DMA,
            pltpu.SemaphoreType.DMA,
            pltpu.SemaphoreType.DMA,
            pltpu.SemaphoreType.DMA,
            pltpu.SemaphoreType.DMA,
        ],
    )
    def _sc_gather(dect_hbm, idx_hbm, out_hbm, idx_v, rows_v,
                   gsem0, gsem1, gsem2, gsem3, wsem0, wsem1, wsem2, wsem3):
        wid = lax.axis_index("s") * 2 + lax.axis_index("c")
        base = wid * _BPW
        gsems = (gsem0, gsem1, gsem2, gsem3)
        wsems = (wsem0, wsem1, wsem2, wsem3)

        pltpu.sync_copy(idx_hbm.at[pl.ds(base, _BPW)], idx_v)

        def gather(c, b):
            return pltpu.async_copy(
                dect_hbm.at[idx_v.at[pl.ds(c * _CH, _CH)]],
                rows_v.at[b], gsems[b])

        pend = [None] * 4
        wpend = [None] * 4
        pend[0] = gather(0, 0)
        pend[1] = gather(1, 1)
        for c in range(_NCH):
            b = c & 3
            if c + 2 < _NCH:
                bn = (c + 2) & 3
                if wpend[bn] is not None:
                    wpend[bn].wait()
                    wpend[bn] = None
                pend[bn] = gather(c + 2, bn)
            pend[b].wait()
            wpend[b] = pltpu.async_copy(
                rows_v.at[b], out_hbm.at[pl.ds(base + c * _CH, _CH)],
                wsems[b])
        for b in range(4):
            if wpend[b] is not None:
                wpend[b].wait()

    return _sc_gather


def kernel(samples, enc_W, enc_b, codebook, dec_W, dec_b):
    x = samples.reshape(_N, _D_IN)
    dec_Wp = jnp.pad(dec_W, ((0, 0), (0, _DP - _D_IN)))
    dec_bp = jnp.pad(dec_b, ((0, _DP - _D_IN),))
    dect, cm2, c2 = _prep_pass(codebook, dec_Wp, dec_bp)
    q, loss = _tc_pass(x, enc_W, enc_b, cm2, c2)
    out = _make_sc_gather()(dect, q)
    return out[:, :_D_IN].reshape(_B, _T, _D_IN), loss[0, 0]


# speedup vs baseline: 1.8345x; 1.8345x over previous
import functools

import jax
import jax.numpy as jnp
from jax import lax
from jax.experimental import pallas as pl
from jax.experimental.pallas import tpu as pltpu
from jax.experimental.pallas import tpu_sc as plsc

_B, _T, _D_IN, _D_LAT, _K = 128, 1024, 96, 32, 512
_N = _B * _T
_TB = 4096
_NT = _N // _TB

_DP = 128
_NW = 32
_BPW = _N // _NW
_CH = 128
_NCH = _BPW // _CH


def _prep_body(cb_ref, decw_ref, decb_ref, dect_ref, cm2_ref, c2_ref):
    cb = cb_ref[...]
    dect_ref[...] = (jnp.dot(cb, decw_ref[...],
                             preferred_element_type=jnp.float32)
                     + decb_ref[...])
    cm2_ref[...] = -2.0 * cb
    c2_ref[...] = jnp.sum(cb * cb, axis=1, keepdims=True)


def _prep_pass(codebook, dec_W, dec_b):
    return pl.pallas_call(
        _prep_body,
        out_shape=[
            jax.ShapeDtypeStruct((_K, _DP), jnp.float32),
            jax.ShapeDtypeStruct((_K, _D_LAT), jnp.float32),
            jax.ShapeDtypeStruct((_K, 1), jnp.float32),
        ],
    )(codebook, dec_W, dec_b.reshape(1, _DP))


def _vq_body(x_ref, encw_ref, encb_ref, cm2_ref, c2_ref, q_ref, loss_ref):
    i = pl.program_id(0)

    @pl.when(i == 0)
    def _init():
        loss_ref[...] = jnp.zeros((1, 1), jnp.float32)

    x = x_ref[...]
    zT = (lax.dot_general(encw_ref[...], x, (((0,), (1,)), ((), ())),
                          preferred_element_type=jnp.float32)
          + encb_ref[...])
    sT = lax.dot_general(cm2_ref[...], zT, (((1,), (0,)), ((), ())),
                         preferred_element_type=jnp.float32)
    dT = sT + c2_ref[...]
    dminT = jnp.min(dT, axis=0, keepdims=True)
    q = jnp.argmin(dT, axis=0).astype(jnp.int32)
    z2T = jnp.sum(zT * zT, axis=0, keepdims=True)
    loss_ref[...] += (jnp.sum(dminT + z2T, keepdims=True)
                      * (1.0 / (_N * _D_LAT)))
    q_ref[...] = q


def _tc_pass(x, enc_W, enc_b, cm2, c2):
    full = lambda i: (0, 0)
    return pl.pallas_call(
        _vq_body,
        grid=(_NT,),
        in_specs=[
            pl.BlockSpec((_TB, _D_IN), lambda i: (i, 0)),
            pl.BlockSpec((_D_IN, _D_LAT), full),
            pl.BlockSpec((_D_LAT, 1), full),
            pl.BlockSpec((_K, _D_LAT), full),
            pl.BlockSpec((_K, 1), full),
        ],
        out_specs=[
            pl.BlockSpec((_TB,), lambda i: (i,)),
            pl.BlockSpec((1, 1), full),
        ],
        out_shape=[
            jax.ShapeDtypeStruct((_N,), jnp.int32),
            jax.ShapeDtypeStruct((1, 1), jnp.float32),
        ],
    )(x, enc_W, enc_b.reshape(_D_LAT, 1), cm2, c2)


@functools.cache
def _make_sc_gather():
    mesh = plsc.VectorSubcoreMesh(core_axis_name="c", subcore_axis_name="s")

    @functools.partial(
        pl.kernel,
        mesh=mesh,
        out_type=jax.ShapeDtypeStruct((_N, _DP), jnp.float32),
        scratch_types=[
            pltpu.VMEM((_BPW,), jnp.int32),
            pltpu.VMEM((4, _CH, _DP), jnp.float32),
            pltpu.VMEM_SHARED((_K, _DP), jnp.float32),
            pltpu.SemaphoreType.DMA,
            pltpu.SemaphoreType.DMA,
            pltpu.SemaphoreType.DMA,
            pltpu.SemaphoreType.DMA,
            pltpu.SemaphoreType.DMA,
            pltpu.SemaphoreType.DMA,
            pltpu.SemaphoreType.DMA,
            pltpu.SemaphoreType.DMA,
        ],
    )
    def _sc_gather(dect_hbm, idx_hbm, out_hbm, idx_v, rows_v, dect_s,
                   gsem0, gsem1, gsem2, gsem3, wsem0, wsem1, wsem2, wsem3):
        sid = lax.axis_index("s")
        wid = sid * 2 + lax.axis_index("c")
        base = wid * _BPW
        gsems = (gsem0, gsem1, gsem2, gsem3)
        wsems = (wsem0, wsem1, wsem2, wsem3)

        @pl.when(sid == 0)
        def _stage():
            pltpu.sync_copy(dect_hbm, dect_s)

        pltpu.sync_copy(idx_hbm.at[pl.ds(base, _BPW)], idx_v)
        plsc.subcore_barrier()

        def gather(c, b):
            return pltpu.async_copy(
                dect_s.at[idx_v.at[pl.ds(c * _CH, _CH)]],
                rows_v.at[b], gsems[b])

        pend = [None] * 4
        wpend = [None] * 4
        pend[0] = gather(0, 0)
        pend[1] = gather(1, 1)
        for c in range(_NCH):
            b = c & 3
            if c + 2 < _NCH:
                bn = (c + 2) & 3
                if wpend[bn] is not None:
                    wpend[bn].wait()
                    wpend[bn] = None
                pend[bn] = gather(c + 2, bn)
            pend[b].wait()
            wpend[b] = pltpu.async_copy(
                rows_v.at[b], out_hbm.at[pl.ds(base + c * _CH, _CH)],
                wsems[b])
        for b in range(4):
            if wpend[b] is not None:
                wpend[b].wait()

    return _sc_gather


def kernel(samples, enc_W, enc_b, codebook, dec_W, dec_b):
    x = samples.reshape(_N, _D_IN)
    dec_Wp = jnp.pad(dec_W, ((0, 0), (0, _DP - _D_IN)))
    dec_bp = jnp.pad(dec_b, ((0, _DP - _D_IN),))
    dect, cm2, c2 = _prep_pass(codebook, dec_Wp, dec_bp)
    q, loss = _tc_pass(x, enc_W, enc_b, cm2, c2)
    out = _make_sc_gather()(dect, q)
    return out[:, :_D_IN].reshape(_B, _T, _D_IN), loss[0, 0]


# speedup vs baseline: 2.2969x; 1.2521x over previous
import functools

import jax
import jax.numpy as jnp
from jax import lax
from jax.experimental import pallas as pl
from jax.experimental.pallas import tpu as pltpu
from jax.experimental.pallas import tpu_sc as plsc

_B, _T, _D_IN, _D_LAT, _K = 128, 1024, 96, 32, 512
_N = _B * _T
_BB = 4
_NT = _B // _BB

_DP = 128
_NW = 32
_BPW = _N // _NW
_CH = 128
_NCH = _BPW // _CH


def _prep_body(cb_ref, decw_ref, decb_ref, dect_ref, cm2_ref, c2_ref):
    cb = cb_ref[...]
    dect_ref[...] = (jnp.dot(cb, decw_ref[...],
                             preferred_element_type=jnp.float32)
                     + decb_ref[...])
    cm2_ref[...] = -2.0 * cb
    c2_ref[...] = jnp.sum(cb * cb, axis=1, keepdims=True)


def _prep_pass(codebook, dec_W, dec_b):
    return pl.pallas_call(
        _prep_body,
        out_shape=[
            jax.ShapeDtypeStruct((_K, _DP), jnp.float32),
            jax.ShapeDtypeStruct((_K, _D_LAT), jnp.float32),
            jax.ShapeDtypeStruct((_K, 1), jnp.float32),
        ],
    )(codebook, dec_W, dec_b.reshape(1, _DP))


def _vq_body(x_ref, encw_ref, encb_ref, cm2_ref, c2_ref, q_ref, loss_ref):
    i = pl.program_id(0)

    @pl.when(i == 0)
    def _init():
        loss_ref[...] = jnp.zeros((1, 1), jnp.float32)

    x2 = x_ref[...]
    encw = encw_ref[...]
    zTs = [lax.dot_general(encw, x2[b * _D_IN:(b + 1) * _D_IN, :],
                           (((0,), (0,)), ((), ())),
                           preferred_element_type=jnp.float32)
           for b in range(_BB)]
    zT = jnp.concatenate(zTs, axis=1) + encb_ref[...]
    sT = lax.dot_general(cm2_ref[...], zT, (((1,), (0,)), ((), ())),
                         preferred_element_type=jnp.float32)
    dT = sT + c2_ref[...]
    dminT = jnp.min(dT, axis=0, keepdims=True)
    q = jnp.argmin(dT, axis=0).astype(jnp.int32)
    z2T = jnp.sum(zT * zT, axis=0, keepdims=True)
    loss_ref[...] += (jnp.sum(dminT + z2T, keepdims=True)
                      * (1.0 / (_N * _D_LAT)))
    q_ref[...] = q


def _tc_pass(xt, enc_W, enc_b, cm2, c2):
    full2 = lambda i: (0, 0)
    full3 = lambda i: (0, 0, 0)
    return pl.pallas_call(
        _vq_body,
        grid=(_NT,),
        in_specs=[
            pl.BlockSpec((_BB * _D_IN, _T), lambda i: (i, 0)),
            pl.BlockSpec((_D_IN, _D_LAT), full2),
            pl.BlockSpec((_D_LAT, 1), full2),
            pl.BlockSpec((_K, _D_LAT), full2),
            pl.BlockSpec((_K, 1), full2),
        ],
        out_specs=[
            pl.BlockSpec((_BB * _T,), lambda i: (i,)),
            pl.BlockSpec((1, 1), full2),
        ],
        out_shape=[
            jax.ShapeDtypeStruct((_N,), jnp.int32),
            jax.ShapeDtypeStruct((1, 1), jnp.float32),
        ],
    )(xt.reshape(_B * _D_IN, _T), enc_W, enc_b.reshape(_D_LAT, 1), cm2, c2)


@functools.cache
def _make_sc_gather():
    mesh = plsc.VectorSubcoreMesh(core_axis_name="c", subcore_axis_name="s")

    @functools.partial(
        pl.kernel,
        mesh=mesh,
        out_type=jax.ShapeDtypeStruct((_N, _DP), jnp.float32),
        scratch_types=[
            pltpu.VMEM((_BPW,), jnp.int32),
            pltpu.VMEM((4, _CH, _DP), jnp.float32),
            pltpu.VMEM_SHARED((_K, _DP), jnp.float32),
            pltpu.SemaphoreType.DMA,
            pltpu.SemaphoreType.DMA,
            pltpu.SemaphoreType.DMA,
            pltpu.SemaphoreType.DMA,
            pltpu.SemaphoreType.DMA,
            pltpu.SemaphoreType.DMA,
            pltpu.SemaphoreType.DMA,
            pltpu.SemaphoreType.DMA,
        ],
    )
    def _sc_gather(dect_hbm, idx_hbm, out_hbm, idx_v, rows_v, dect_s,
                   gsem0, gsem1, gsem2, gsem3, wsem0, wsem1, wsem2, wsem3):
        sid = lax.axis_index("s")
        wid = sid * 2 + lax.axis_index("c")
        base = wid * _BPW
        gsems = (gsem0, gsem1, gsem2, gsem3)
        wsems = (wsem0, wsem1, wsem2, wsem3)

        @pl.when(sid == 0)
        def _stage():
            pltpu.sync_copy(dect_hbm, dect_s)

        pltpu.sync_copy(idx_hbm.at[pl.ds(base, _BPW)], idx_v)
        plsc.subcore_barrier()

        def gather(c, b):
            return pltpu.async_copy(
                dect_s.at[idx_v.at[pl.ds(c * _CH, _CH)]],
                rows_v.at[b], gsems[b])

        pend = [None] * 4
        wpend = [None] * 4
        pend[0] = gather(0, 0)
        pend[1] = gather(1, 1)
        for c in range(_NCH):
            b = c & 3
            if c + 2 < _NCH:
                bn = (c + 2) & 3
                if wpend[bn] is not None:
                    wpend[bn].wait()
                    wpend[bn] = None
                pend[bn] = gather(c + 2, bn)
            pend[b].wait()
            wpend[b] = pltpu.async_copy(
                rows_v.at[b], out_hbm.at[pl.ds(base + c * _CH, _CH)],
                wsems[b])
        for b in range(4):
            if wpend[b] is not None:
                wpend[b].wait()

    return _sc_gather


def kernel(samples, enc_W, enc_b, codebook, dec_W, dec_b):
    xt = jnp.swapaxes(samples, 1, 2)
    dec_Wp = jnp.pad(dec_W, ((0, 0), (0, _DP - _D_IN)))
    dec_bp = jnp.pad(dec_b, ((0, _DP - _D_IN),))
    dect, cm2, c2 = _prep_pass(codebook, dec_Wp, dec_bp)
    q, loss = _tc_pass(xt, enc_W, enc_b, cm2, c2)
    out = _make_sc_gather()(dect, q)
    return out[:, :_D_IN].reshape(_B, _T, _D_IN), loss[0, 0]
